# baseline (speedup 1.0000x reference)
import functools

import jax
import jax.numpy as jnp
from jax import lax
from jax.experimental import pallas as pl
from jax.experimental.pallas import tpu as pltpu
from jax.experimental.pallas import tpu_sc as plsc

CHUNK = 128
BIG = 2
EDGE_SLICES = 16
HALF = 5120


def _mm_body(x_ref, w_ref, b_ref, y_ref, ys_ref, *, d_out):
    both = jnp.dot(x_ref[...], w_ref[...], preferred_element_type=jnp.float32)
    y_ref[...] = both[:, :d_out]
    ys_ref[...] = both[:, d_out:] + b_ref[...]


def _matmuls(x, w_cat, b_row, d_out, bm):
    n = x.shape[0]
    d_in = x.shape[1]
    return pl.pallas_call(
        functools.partial(_mm_body, d_out=d_out),
        grid=(n // bm,),
        in_specs=[
            pl.BlockSpec((bm, d_in), lambda i: (i, 0)),
            pl.BlockSpec((d_in, 2 * d_out), lambda i: (0, 0)),
            pl.BlockSpec((1, d_out), lambda i: (0, 0)),
        ],
        out_specs=[
            pl.BlockSpec((bm, d_out), lambda i: (i, 0)),
            pl.BlockSpec((bm, d_out), lambda i: (i, 0)),
        ],
        out_shape=[
            jax.ShapeDtypeStruct((n, d_out), jnp.float32),
            jax.ShapeDtypeStruct((n, d_out), jnp.float32),
        ],
    )(x, w_cat, b_row)


def _make_sc_scatter(n, d, n_chunks):
    mesh = plsc.VectorSubcoreMesh(core_axis_name="c", subcore_axis_name="s")
    hi_rows = n - HALF
    lo_rpt = HALF // 16
    hi_rpt = 304
    hi_last = hi_rows - 15 * hi_rpt

    @functools.partial(
        pl.kernel,
        mesh=mesh,
        out_type=jax.ShapeDtypeStruct((n, d), jnp.float32),
        scratch_types=[
            pltpu.VMEM((1, n_chunks * CHUNK), jnp.int32),
            pltpu.VMEM((n_chunks, CHUNK), jnp.int32),
            pltpu.VMEM((BIG, CHUNK), jnp.int32),
            pltpu.VMEM((BIG * CHUNK, d), jnp.float32),
            pltpu.VMEM_SHARED((HALF + 8, d), jnp.float32),
            pltpu.SemaphoreType.DMA,
        ],
    )
    def sc_scatter(y_hbm, src_hbm, dst_hbm, yself_hbm,
                   out_hbm, src_v, dst_v, idx_big, rows_v, acc, sem):
        cid = lax.axis_index("c")
        sid = lax.axis_index("s")
        lo = cid * HALF

        pltpu.sync_copy(src_hbm.at[sid], src_v)
        pltpu.sync_copy(dst_hbm.at[sid], dst_v)

        @pl.when(cid == 0)
        def _():
            pltpu.sync_copy(yself_hbm.at[pl.ds(sid * lo_rpt, lo_rpt)],
                            acc.at[pl.ds(sid * lo_rpt, lo_rpt)])

        @pl.when((cid == 1) & (sid < 15))
        def _():
            pltpu.sync_copy(yself_hbm.at[pl.ds(HALF + sid * hi_rpt, hi_rpt)],
                            acc.at[pl.ds(sid * hi_rpt, hi_rpt)])

        @pl.when((cid == 1) & (sid == 15))
        def _():
            pltpu.sync_copy(yself_hbm.at[pl.ds(HALF + 15 * hi_rpt, hi_last)],
                            acc.at[pl.ds(15 * hi_rpt, hi_last)])

        plsc.subcore_barrier()

        def body(jb, carry):
            cp = pltpu.async_copy(
                y_hbm.at[src_v.at[0, pl.ds(jb * BIG * CHUNK, BIG * CHUNK)]],
                rows_v, sem)
            for r in range(BIG):
                for k in range(CHUNK // 16):
                    dstk = dst_v[jb * BIG + r, pl.ds(k * 16, 16)]
                    local = dstk - lo
                    ok = (local >= 0) & (local < HALF)
                    idx_big[r, pl.ds(k * 16, 16)] = jnp.where(ok, local, HALF)
            cp.wait()
            for r in range(BIG):
                pltpu.sync_copy(rows_v.at[pl.ds(r * CHUNK, CHUNK)],
                                acc.at[idx_big.at[r]], add=True)
            return carry

        lax.fori_loop(0, n_chunks // BIG, body, 0, unroll=False)

        plsc.subcore_barrier()

        @pl.when(cid == 0)
        def _():
            pltpu.sync_copy(acc.at[pl.ds(sid * lo_rpt, lo_rpt)],
                            out_hbm.at[pl.ds(sid * lo_rpt, lo_rpt)])

        @pl.when((cid == 1) & (sid < 15))
        def _():
            pltpu.sync_copy(acc.at[pl.ds(sid * hi_rpt, hi_rpt)],
                            out_hbm.at[pl.ds(HALF + sid * hi_rpt, hi_rpt)])

        @pl.when((cid == 1) & (sid == 15))
        def _():
            pltpu.sync_copy(acc.at[pl.ds(15 * hi_rpt, hi_last)],
                            out_hbm.at[pl.ds(HALF + 15 * hi_rpt, hi_last)])

    return sc_scatter


def kernel(x, edge_index, W, W_self, b):
    n, d_in = x.shape
    d_out = W.shape[1]
    e = edge_index.shape[1]

    w_cat = jnp.concatenate([W, W_self], axis=1)
    y, yself = _matmuls(x, w_cat, b.reshape(1, d_out), d_out, bm=2000)

    grain = EDGE_SLICES * CHUNK * BIG
    e_pad = ((e + grain - 1) // grain) * grain
    n_chunks = e_pad // (EDGE_SLICES * CHUNK)
    src2 = jnp.pad(edge_index[0], (0, e_pad - e)).reshape(
        EDGE_SLICES, 1, n_chunks * CHUNK)
    dst2 = jnp.pad(edge_index[1], (0, e_pad - e),
                   constant_values=2 * n).reshape(EDGE_SLICES, n_chunks, CHUNK)

    return _make_sc_scatter(n, d_out, n_chunks)(y, src2, dst2, yself)


# speedup vs baseline: 1.2705x; 1.2705x over previous
import functools

import jax
import jax.numpy as jnp
from jax import lax
from jax.experimental import pallas as pl
from jax.experimental.pallas import tpu as pltpu
from jax.experimental.pallas import tpu_sc as plsc

CHUNK = 128
EDGE_SLICES = 16
HALF = 5120


def _mm_body(x_ref, w_ref, b_ref, y_ref, ys_ref, *, d_out):
    both = jnp.dot(x_ref[...], w_ref[...], preferred_element_type=jnp.float32)
    y_ref[...] = both[:, :d_out]
    ys_ref[...] = both[:, d_out:] + b_ref[...]


def _matmuls(x, w_cat, b_row, d_out, bm):
    n = x.shape[0]
    d_in = x.shape[1]
    return pl.pallas_call(
        functools.partial(_mm_body, d_out=d_out),
        grid=(n // bm,),
        in_specs=[
            pl.BlockSpec((bm, d_in), lambda i: (i, 0)),
            pl.BlockSpec((d_in, 2 * d_out), lambda i: (0, 0)),
            pl.BlockSpec((1, d_out), lambda i: (0, 0)),
        ],
        out_specs=[
            pl.BlockSpec((bm, d_out), lambda i: (i, 0)),
            pl.BlockSpec((bm, d_out), lambda i: (i, 0)),
        ],
        out_shape=[
            jax.ShapeDtypeStruct((n, d_out), jnp.float32),
            jax.ShapeDtypeStruct((n, d_out), jnp.float32),
        ],
    )(x, w_cat, b_row)


def _make_sc_scatter(n, d, n_chunks):
    mesh = plsc.VectorSubcoreMesh(core_axis_name="c", subcore_axis_name="s")
    hi_rows = n - HALF
    lo_rpt = HALF // 16
    hi_rpt = 304
    hi_last = hi_rows - 15 * hi_rpt

    @functools.partial(
        pl.kernel,
        mesh=mesh,
        out_type=jax.ShapeDtypeStruct((n, d), jnp.float32),
        scratch_types=[
            pltpu.VMEM((n_chunks, CHUNK), jnp.int32),
            pltpu.VMEM((n_chunks, CHUNK), jnp.int32),
            pltpu.VMEM((1, CHUNK), jnp.int32),
            pltpu.VMEM((CHUNK, d), jnp.float32),
            pltpu.VMEM_SHARED((HALF + 8, d), jnp.float32),
            pltpu.SemaphoreType.DMA,
        ],
    )
    def sc_scatter(y_hbm, src_hbm, dst_hbm, yself_hbm,
                   out_hbm, src_v, dst_v, idx_row, rows_v, acc, sem):
        cid = lax.axis_index("c")
        sid = lax.axis_index("s")
        lo = cid * HALF

        pltpu.sync_copy(src_hbm.at[sid], src_v)
        pltpu.sync_copy(dst_hbm.at[sid], dst_v)

        @pl.when(cid == 0)
        def _():
            pltpu.sync_copy(yself_hbm.at[pl.ds(sid * lo_rpt, lo_rpt)],
                            acc.at[pl.ds(sid * lo_rpt, lo_rpt)])

        @pl.when((cid == 1) & (sid < 15))
        def _():
            pltpu.sync_copy(yself_hbm.at[pl.ds(HALF + sid * hi_rpt, hi_rpt)],
                            acc.at[pl.ds(sid * hi_rpt, hi_rpt)])

        @pl.when((cid == 1) & (sid == 15))
        def _():
            pltpu.sync_copy(yself_hbm.at[pl.ds(HALF + 15 * hi_rpt, hi_last)],
                            acc.at[pl.ds(15 * hi_rpt, hi_last)])

        plsc.subcore_barrier()

        def body(j, carry):
            cp = pltpu.async_copy(y_hbm.at[src_v.at[j]], rows_v, sem)
            for k in range(CHUNK // 16):
                dstk = dst_v[j, pl.ds(k * 16, 16)]
                local = dstk - lo
                ok = (local >= 0) & (local < HALF)
                idx_row[0, pl.ds(k * 16, 16)] = jnp.where(ok, local, HALF)
            cp.wait()
            pltpu.sync_copy(rows_v, acc.at[idx_row.at[0]], add=True)
            return carry

        lax.fori_loop(0, n_chunks, body, 0, unroll=False)

        plsc.subcore_barrier()

        @pl.when(cid == 0)
        def _():
            pltpu.sync_copy(acc.at[pl.ds(sid * lo_rpt, lo_rpt)],
                            out_hbm.at[pl.ds(sid * lo_rpt, lo_rpt)])

        @pl.when((cid == 1) & (sid < 15))
        def _():
            pltpu.sync_copy(acc.at[pl.ds(sid * hi_rpt, hi_rpt)],
                            out_hbm.at[pl.ds(HALF + sid * hi_rpt, hi_rpt)])

        @pl.when((cid == 1) & (sid == 15))
        def _():
            pltpu.sync_copy(acc.at[pl.ds(15 * hi_rpt, hi_last)],
                            out_hbm.at[pl.ds(HALF + 15 * hi_rpt, hi_last)])

    return sc_scatter


def kernel(x, edge_index, W, W_self, b):
    n, d_in = x.shape
    d_out = W.shape[1]
    e = edge_index.shape[1]

    w_cat = jnp.concatenate([W, W_self], axis=1)
    y, yself = _matmuls(x, w_cat, b.reshape(1, d_out), d_out, bm=2000)

    grain = EDGE_SLICES * CHUNK
    e_pad = ((e + grain - 1) // grain) * grain
    n_chunks = e_pad // grain
    src2 = jnp.pad(edge_index[0], (0, e_pad - e)).reshape(
        EDGE_SLICES, n_chunks, CHUNK)
    dst2 = jnp.pad(edge_index[1], (0, e_pad - e),
                   constant_values=2 * n).reshape(EDGE_SLICES, n_chunks, CHUNK)

    return _make_sc_scatter(n, d_out, n_chunks)(y, src2, dst2, yself)
